# baseline (speedup 1.0000x reference)
import functools
import math

import jax
import jax.numpy as jnp
import numpy as np
from jax import lax
from jax.experimental import pallas as pl
from jax.experimental.pallas import tpu as pltpu
from jax.experimental.pallas import tpu_sc as plsc

_IRR_NODE = ((32, 0), (16, 1), (8, 2))
_IRR_EDGE = ((1, 0), (1, 1), (1, 2))
_INS = ((0, 0, 0), (0, 1, 1), (0, 2, 2), (1, 0, 1), (1, 1, 0), (1, 1, 2),
        (1, 2, 1), (2, 0, 2), (2, 1, 1), (2, 2, 0), (2, 2, 2))


def _su2_cg_coeff(j1, m1, j2, m2, j3, m3):
    if m3 != m1 + m2:
        return 0.0
    f = math.factorial
    vmin = int(max(-j1 + j2 + m3, -j1 + m1, 0))
    vmax = int(min(j2 + j3 + m1, j3 - j1 + j2, j3 + m3))
    C = math.sqrt((2 * j3 + 1) * f(j3 + j1 - j2) * f(j3 - j1 + j2) * f(j1 + j2 - j3) * f(j3 + m3) * f(j3 - m3)
                  / (f(j1 + j2 + j3 + 1) * f(j1 - m1) * f(j1 + m1) * f(j2 - m2) * f(j2 + m2)))
    S = 0.0
    for v in range(vmin, vmax + 1):
        S += (-1.0) ** (v + j2 + m2) * f(j2 + j3 + m1 - v) * f(j1 - m1 + v) / (
            f(v) * f(j3 - j1 + j2 - v) * f(j3 + m3 - v) * f(v + j1 - j2 - m3))
    return C * S


def _real_q(l):
    q = np.zeros((2 * l + 1, 2 * l + 1), dtype=np.complex128)
    for m in range(-l, 0):
        q[l + m, l + abs(m)] = 1.0 / math.sqrt(2.0)
        q[l + m, l - abs(m)] = -1j / math.sqrt(2.0)
    q[l, l] = 1.0
    for m in range(1, l + 1):
        q[l + m, l + abs(m)] = (-1) ** m / math.sqrt(2.0)
        q[l + m, l - abs(m)] = 1j * ((-1) ** m) / math.sqrt(2.0)
    return ((-1j) ** l) * q


def _wigner3j(l1, l2, l3):
    C = np.zeros((2 * l1 + 1, 2 * l2 + 1, 2 * l3 + 1))
    for m1 in range(-l1, l1 + 1):
        for m2 in range(-l2, l2 + 1):
            m3 = m1 + m2
            if abs(m3) <= l3:
                C[l1 + m1, l2 + m2, l3 + m3] = _su2_cg_coeff(l1, m1, l2, m2, l3, m3)
    Q1, Q2, Q3 = _real_q(l1), _real_q(l2), _real_q(l3)
    Cr = np.real(np.einsum('ij,kl,mn,ikn->jlm', Q1, Q2, np.conj(Q3.T), C.astype(np.complex128)))
    n = np.linalg.norm(Cr)
    return (Cr / n).astype(np.float32) if n > 0 else Cr.astype(np.float32)


_C3J = {}
for _a, _b, _c in _INS:
    _k = (_IRR_NODE[_a][1], _IRR_EDGE[_b][1], _IRR_NODE[_c][1])
    if _k not in _C3J:
        _C3J[_k] = _wigner3j(*_k)

_PATHC = []
for (_i1, _i2, _io) in _INS:
    _fan = sum(_IRR_NODE[a][0] * _IRR_EDGE[b][0] for (a, b, c) in _INS if c == _io)
    _PATHC.append(math.sqrt((2 * _IRR_NODE[_io][1] + 1) / _fan))

_NODE_OFF = [0, 32, 80]
_EDGE_OFF = [0, 1, 4]
_DIM_NODE = 120
_W_OFF = []
_off = 0
for (_i1, _i2, _io) in _INS:
    _W_OFF.append(_off)
    _off += _IRR_NODE[_i1][0] * _IRR_NODE[_io][0]
_W_TOT = _off

_NNZ = []
_R_MATS = []
_S_MATS = []
for _idx, (_i1, _i2, _io) in enumerate(_INS):
    _mul1, _l1 = _IRR_NODE[_i1]
    _l2 = _IRR_EDGE[_i2][1]
    _mulo, _lo = _IRR_NODE[_io]
    _C = _C3J[(_l1, _l2, _lo)]
    _per_k = []
    for _kk in range(2 * _lo + 1):
        _lst = []
        for _ii in range(2 * _l1 + 1):
            for _jj in range(2 * _l2 + 1):
                _c = float(_C[_ii, _jj, _kk])
                if _c != 0.0:
                    _lst.append((_ii, _jj, _c))
        _per_k.append(_lst)
    _NNZ.append(_per_k)
    _R = np.zeros((_mul1, _mul1 * _mulo), np.float32)
    _S = np.zeros((_mul1 * _mulo, _mulo), np.float32)
    for _u in range(_mul1):
        for _w in range(_mulo):
            _R[_u, _u * _mulo + _w] = 1.0
            _S[_u * _mulo + _w, _w] = _PATHC[_idx]
    _R_MATS.append(_R)
    _S_MATS.append(_S)

_BE = 256
_CH = 128
_FPAD = 128

_I2_GROUPS = [[i for i, ins in enumerate(_INS) if ins[1] == g] for g in range(3)]
_TALL_OFF = {}
_G_MATS = []
_cursor = 0
for _g in range(3):
    _d2 = 2 * _IRR_EDGE[_g][1] + 1
    _cols = 0
    for _idx in _I2_GROUPS[_g]:
        _i1 = _INS[_idx][0]
        _lo = _IRR_NODE[_INS[_idx][2]][1]
        _TALL_OFF[_idx] = _cursor + _cols
        _cols += (2 * _lo + 1) * _IRR_NODE[_i1][0]
    _G = np.zeros((_d2 * _FPAD, _cols), np.float32)
    _coff = 0
    for _idx in _I2_GROUPS[_g]:
        _i1, _i2, _io = _INS[_idx]
        _mul1, _l1 = _IRR_NODE[_i1]
        _lo = _IRR_NODE[_io][1]
        _C = _C3J[(_l1, _IRR_EDGE[_i2][1], _lo)]
        for _kk in range(2 * _lo + 1):
            for _ii in range(2 * _l1 + 1):
                for _jl in range(_d2):
                    _c = float(_C[_ii, _jl, _kk])
                    if _c != 0.0:
                        for _u in range(_mul1):
                            _G[_jl * _FPAD + _NODE_OFF[_i1] + _ii * _mul1 + _u,
                               _coff + _kk * _mul1 + _u] = _c
        _coff += (2 * _lo + 1) * _mul1
    _G_MATS.append(_G)
    _cursor += _cols
_TALL_DIM = _cursor

_BT_ROWS, _BT_COLS = [], []
for _gi, (_mul, _l) in enumerate(_IRR_NODE):
    _d = 2 * _l + 1
    for _kk in range(_d):
        for _u in range(_mul):
            for _v in range(_mul):
                _BT_ROWS.append(_NODE_OFF[_gi] + _kk * _mul + _u)
                _BT_COLS.append(_NODE_OFF[_gi] + _v * _d + _kk)
_BT_ROWS = np.asarray(_BT_ROWS, np.int32)
_BT_COLS = np.asarray(_BT_COLS, np.int32)

def _build_bt(lw0, lw1, lw2):
    vals = []
    for lw, (mul, l) in zip((lw0, lw1, lw2), _IRR_NODE):
        d = 2 * l + 1
        vals.append(jnp.tile(lw[None, :, :] / np.float32(math.sqrt(mul)), (d, 1, 1)).reshape(-1))
    vals = jnp.concatenate(vals)
    return jnp.zeros((_FPAD, _FPAD), jnp.float32).at[_BT_ROWS, _BT_COLS].set(vals)


def _silu(x):
    return x / (1.0 + jnp.exp(-x))


def _tc_body(x1_ref, sh_ref, er_ref, w1_ref, b1_ref, w2_ref, b2_ref, w3_ref,
             b3_ref, bt_ref, g0_ref, g1_ref, g2_ref, *rest):
    out_ref = rest[-1]
    rs = rest[:-1]
    f32 = jnp.float32
    bf16 = jnp.bfloat16

    er = er_ref[...]
    h = _silu(jnp.dot(er.astype(bf16), w1_ref[...], preferred_element_type=f32) + b1_ref[0:1, :])
    h = _silu(jnp.dot(h.astype(bf16), w2_ref[...], preferred_element_type=f32) + b2_ref[0:1, :])
    w = jnp.dot(h.astype(bf16), w3_ref[...], preferred_element_type=f32) + b3_ref[0:1, :]

    sh = sh_ref[...].astype(bf16)
    x1 = x1_ref[...].astype(bf16)

    zs = [x1 * sh[:, j:j + 1] for j in range(9)]
    g_refs = (g0_ref, g1_ref, g2_ref)
    tparts = []
    for g, (lo_j, hi_j) in enumerate(((0, 1), (1, 4), (4, 9))):
        z = zs[lo_j] if hi_j - lo_j == 1 else jnp.concatenate(zs[lo_j:hi_j], axis=1)
        tparts.append(jnp.dot(z, g_refs[g][...], preferred_element_type=f32))
    tall = jnp.concatenate(tparts, axis=1)

    parts = [[None] * (2 * l + 1) for (_, l) in _IRR_NODE]
    for idx, (i1, i2, io) in enumerate(_INS):
        mul1, l1 = _IRR_NODE[i1]
        mulo, lo = _IRR_NODE[io]
        nw = mul1 * mulo
        r_ref, s_ref = rs[2 * idx], rs[2 * idx + 1]
        wsl = w[:, _W_OFF[idx]:_W_OFF[idx] + nw]
        for k in range(2 * lo + 1):
            tk = tall[:, _TALL_OFF[idx] + k * mul1:_TALL_OFF[idx] + (k + 1) * mul1]
            texp = jnp.dot(tk.astype(bf16), r_ref[...], preferred_element_type=f32)
            part = jnp.dot((wsl * texp).astype(bf16), s_ref[...], preferred_element_type=f32)
            parts[io][k] = part if parts[io][k] is None else parts[io][k] + part

    cols = []
    for gi in range(3):
        cols.extend(parts[gi])
    cols.append(jnp.zeros((x1.shape[0], _FPAD - _DIM_NODE), f32))
    msg_t = jnp.concatenate(cols, axis=1)
    out_ref[...] = jnp.dot(msg_t.astype(bf16), bt_ref[...], preferred_element_type=f32)


def _tc_messages(x1t, sh_pad, er_pad, w1t, b1r, w2t, b2r, w3t, b3r, bt, consts):
    epad = x1t.shape[0]
    grid = (epad // _BE,)
    edge_spec = lambda width: pl.BlockSpec((_BE, width), lambda i: (i, 0))
    full = lambda a: pl.BlockSpec(a.shape, lambda i: (0, 0))
    gmats = [jnp.asarray(g, jnp.bfloat16) for g in _G_MATS]
    in_specs = [edge_spec(_FPAD), edge_spec(16), edge_spec(8),
                full(w1t), full(b1r), full(w2t), full(b2r), full(w3t),
                full(b3r), full(bt)] + [full(g) for g in gmats] + [full(c) for c in consts]
    return pl.pallas_call(
        _tc_body,
        grid=grid,
        in_specs=in_specs,
        out_specs=pl.BlockSpec((_BE, _FPAD), lambda i: (i, 0)),
        out_shape=jax.ShapeDtypeStruct((epad, _FPAD), jnp.float32),
    )(x1t, sh_pad, er_pad, w1t, b1r, w2t, b2r, w3t, b3r, bt, *gmats, *consts)


def _sc_gather(table, src_pad):
    epad = src_pad.shape[0]
    mesh = plsc.VectorSubcoreMesh(core_axis_name="c", subcore_axis_name="s")
    rows_pt = epad // 32
    nch = rows_pt // _CH

    @functools.partial(
        pl.kernel, mesh=mesh,
        out_type=jax.ShapeDtypeStruct((epad, _FPAD), jnp.float32),
        scratch_types=[pltpu.VMEM((_CH,), jnp.int32),
                       pltpu.VMEM((_CH, _FPAD), jnp.float32),
                       pltpu.SemaphoreType.DMA],
    )
    def k(table_hbm, src_hbm, out_hbm, idx_v, rows_v, sem):
        wid = lax.axis_index("s") * 2 + lax.axis_index("c")

        def step(i, carry):
            base = wid * rows_pt + i * _CH
            pltpu.sync_copy(src_hbm.at[pl.ds(base, _CH)], idx_v)
            pltpu.async_copy(table_hbm.at[idx_v], rows_v, sem).wait()
            pltpu.sync_copy(rows_v, out_hbm.at[pl.ds(base, _CH)])
            return carry

        lax.fori_loop(0, nch, step, 0)

    return k(table, src_pad)


def _sc_scatter(msgs, dst_pad, nf_pad):
    epad = msgs.shape[0]
    npad = nf_pad.shape[0]
    mesh = plsc.VectorSubcoreMesh(core_axis_name="c", subcore_axis_name="s")
    rows_pt = epad // 16
    nch = rows_pt // _CH
    init_pt = npad // 16

    @functools.partial(
        pl.kernel, mesh=mesh,
        out_type=jax.ShapeDtypeStruct((npad, _FPAD), jnp.float32),
        scratch_types=[pltpu.VMEM((_CH, _FPAD), jnp.float32),
                       pltpu.VMEM((_CH,), jnp.int32),
                       pltpu.VMEM_SHARED((npad, _FPAD), jnp.float32)],
    )
    def k(msg_hbm, dst_hbm, nf_hbm, out_hbm, buf_v, idx_v, acc_sh):
        cid = lax.axis_index("c")
        sid = lax.axis_index("s")

        @pl.when(cid == 0)
        def _():
            pltpu.sync_copy(nf_hbm.at[pl.ds(sid * init_pt, init_pt)],
                            acc_sh.at[pl.ds(sid * init_pt, init_pt)])
            plsc.subcore_barrier()

            def step(i, carry):
                base = sid * rows_pt + i * _CH
                pltpu.sync_copy(dst_hbm.at[pl.ds(base, _CH)], idx_v)
                pltpu.sync_copy(msg_hbm.at[pl.ds(base, _CH)], buf_v)
                pltpu.sync_copy(buf_v, acc_sh.at[idx_v], add=True)
                return carry

            lax.fori_loop(0, nch, step, 0)
            plsc.subcore_barrier()
            pltpu.sync_copy(acc_sh.at[pl.ds(sid * init_pt, init_pt)],
                            out_hbm.at[pl.ds(sid * init_pt, init_pt)])

    return k(msgs, dst_pad, nf_pad)


def kernel(node_features, edge_index, edge_sh, edge_radial, W1, b1, W2, b2,
           W3, b3, lw0, lw1, lw2):
    N = node_features.shape[0]
    E = edge_sh.shape[0]
    epad = ((E + 4095) // 4096) * 4096
    npad = ((N + 2047) // 2048) * 2048

    segs = []
    for gi, (mul, l) in enumerate(_IRR_NODE):
        d = 2 * l + 1
        p = node_features[:, _NODE_OFF[gi]:_NODE_OFF[gi] + mul * d]
        segs.append(p.reshape(N, mul, d).transpose(0, 2, 1).reshape(N, mul * d))
    node_t = jnp.concatenate(segs + [jnp.zeros((N, _FPAD - _DIM_NODE), jnp.float32)], axis=1)

    src = edge_index[0].astype(jnp.int32)
    dst = edge_index[1].astype(jnp.int32)
    src_pad = jnp.zeros((epad,), jnp.int32).at[:E].set(src)
    dst_pad = jnp.zeros((epad,), jnp.int32).at[:E].set(dst)
    sh_pad = jnp.zeros((epad, 16), jnp.float32).at[:E, :9].set(edge_sh)
    er_pad = jnp.zeros((epad, 8), jnp.float32).at[:E, :].set(edge_radial)

    bf16 = jnp.bfloat16
    w1t = W1.T.astype(bf16)
    w2t = W2.T.astype(bf16)
    w3t = W3.T.astype(bf16)
    b1r = jnp.tile(b1[None, :], (8, 1))
    b2r = jnp.tile(b2[None, :], (8, 1))
    b3r = jnp.tile(b3[None, :], (8, 1))
    bt = _build_bt(lw0, lw1, lw2).astype(bf16)

    consts = []
    for r, s in zip(_R_MATS, _S_MATS):
        consts.append(jnp.asarray(r, bf16))
        consts.append(jnp.asarray(s, bf16))

    x1t = _sc_gather(node_t, src_pad)
    msgs = _tc_messages(x1t, sh_pad, er_pad, w1t, b1r, w2t, b2r, w3t, b3r, bt, consts)

    nf_pad = jnp.zeros((npad, _FPAD), jnp.float32).at[:N, :_DIM_NODE].set(node_features)
    outp = _sc_scatter(msgs, dst_pad, nf_pad)
    return outp[:N, :_DIM_NODE]


# speedup vs baseline: 1.1927x; 1.1927x over previous
import functools
import math

import jax
import jax.numpy as jnp
import numpy as np
from jax import lax
from jax.experimental import pallas as pl
from jax.experimental.pallas import tpu as pltpu
from jax.experimental.pallas import tpu_sc as plsc

_IRR_NODE = ((32, 0), (16, 1), (8, 2))
_IRR_EDGE = ((1, 0), (1, 1), (1, 2))
_INS = ((0, 0, 0), (0, 1, 1), (0, 2, 2), (1, 0, 1), (1, 1, 0), (1, 1, 2),
        (1, 2, 1), (2, 0, 2), (2, 1, 1), (2, 2, 0), (2, 2, 2))


def _su2_cg_coeff(j1, m1, j2, m2, j3, m3):
    if m3 != m1 + m2:
        return 0.0
    f = math.factorial
    vmin = int(max(-j1 + j2 + m3, -j1 + m1, 0))
    vmax = int(min(j2 + j3 + m1, j3 - j1 + j2, j3 + m3))
    C = math.sqrt((2 * j3 + 1) * f(j3 + j1 - j2) * f(j3 - j1 + j2) * f(j1 + j2 - j3) * f(j3 + m3) * f(j3 - m3)
                  / (f(j1 + j2 + j3 + 1) * f(j1 - m1) * f(j1 + m1) * f(j2 - m2) * f(j2 + m2)))
    S = 0.0
    for v in range(vmin, vmax + 1):
        S += (-1.0) ** (v + j2 + m2) * f(j2 + j3 + m1 - v) * f(j1 - m1 + v) / (
            f(v) * f(j3 - j1 + j2 - v) * f(j3 + m3 - v) * f(v + j1 - j2 - m3))
    return C * S


def _real_q(l):
    q = np.zeros((2 * l + 1, 2 * l + 1), dtype=np.complex128)
    for m in range(-l, 0):
        q[l + m, l + abs(m)] = 1.0 / math.sqrt(2.0)
        q[l + m, l - abs(m)] = -1j / math.sqrt(2.0)
    q[l, l] = 1.0
    for m in range(1, l + 1):
        q[l + m, l + abs(m)] = (-1) ** m / math.sqrt(2.0)
        q[l + m, l - abs(m)] = 1j * ((-1) ** m) / math.sqrt(2.0)
    return ((-1j) ** l) * q


def _wigner3j(l1, l2, l3):
    C = np.zeros((2 * l1 + 1, 2 * l2 + 1, 2 * l3 + 1))
    for m1 in range(-l1, l1 + 1):
        for m2 in range(-l2, l2 + 1):
            m3 = m1 + m2
            if abs(m3) <= l3:
                C[l1 + m1, l2 + m2, l3 + m3] = _su2_cg_coeff(l1, m1, l2, m2, l3, m3)
    Q1, Q2, Q3 = _real_q(l1), _real_q(l2), _real_q(l3)
    Cr = np.real(np.einsum('ij,kl,mn,ikn->jlm', Q1, Q2, np.conj(Q3.T), C.astype(np.complex128)))
    n = np.linalg.norm(Cr)
    return (Cr / n).astype(np.float32) if n > 0 else Cr.astype(np.float32)


_C3J = {}
for _a, _b, _c in _INS:
    _k = (_IRR_NODE[_a][1], _IRR_EDGE[_b][1], _IRR_NODE[_c][1])
    if _k not in _C3J:
        _C3J[_k] = _wigner3j(*_k)

_PATHC = []
for (_i1, _i2, _io) in _INS:
    _fan = sum(_IRR_NODE[a][0] * _IRR_EDGE[b][0] for (a, b, c) in _INS if c == _io)
    _PATHC.append(math.sqrt((2 * _IRR_NODE[_io][1] + 1) / _fan))

_NODE_OFF = [0, 32, 80]
_EDGE_OFF = [0, 1, 4]
_DIM_NODE = 120
_W_OFF = []
_off = 0
for (_i1, _i2, _io) in _INS:
    _W_OFF.append(_off)
    _off += _IRR_NODE[_i1][0] * _IRR_NODE[_io][0]
_W_TOT = _off

_NNZ = []
_R_MATS = []
_S_MATS = []
for _idx, (_i1, _i2, _io) in enumerate(_INS):
    _mul1, _l1 = _IRR_NODE[_i1]
    _l2 = _IRR_EDGE[_i2][1]
    _mulo, _lo = _IRR_NODE[_io]
    _C = _C3J[(_l1, _l2, _lo)]
    _per_k = []
    for _kk in range(2 * _lo + 1):
        _lst = []
        for _ii in range(2 * _l1 + 1):
            for _jj in range(2 * _l2 + 1):
                _c = float(_C[_ii, _jj, _kk])
                if _c != 0.0:
                    _lst.append((_ii, _jj, _c))
        _per_k.append(_lst)
    _NNZ.append(_per_k)
    _R = np.zeros((_mul1, _mul1 * _mulo), np.float32)
    _S = np.zeros((_mul1 * _mulo, _mulo), np.float32)
    for _u in range(_mul1):
        for _w in range(_mulo):
            _R[_u, _u * _mulo + _w] = 1.0
            _S[_u * _mulo + _w, _w] = _PATHC[_idx]
    _R_MATS.append(_R)
    _S_MATS.append(_S)

_BE = 512
_CH = 128
_FPAD = 128

_I2_GROUPS = [[i for i, ins in enumerate(_INS) if ins[1] == g] for g in range(3)]
_TALL_OFF = {}
_G_MATS = []
_cursor = 0
for _g in range(3):
    _d2 = 2 * _IRR_EDGE[_g][1] + 1
    _cols = 0
    for _idx in _I2_GROUPS[_g]:
        _i1 = _INS[_idx][0]
        _lo = _IRR_NODE[_INS[_idx][2]][1]
        _TALL_OFF[_idx] = _cursor + _cols
        _cols += (2 * _lo + 1) * _IRR_NODE[_i1][0]
    _G = np.zeros((_d2 * _FPAD, _cols), np.float32)
    _coff = 0
    for _idx in _I2_GROUPS[_g]:
        _i1, _i2, _io = _INS[_idx]
        _mul1, _l1 = _IRR_NODE[_i1]
        _lo = _IRR_NODE[_io][1]
        _C = _C3J[(_l1, _IRR_EDGE[_i2][1], _lo)]
        for _kk in range(2 * _lo + 1):
            for _ii in range(2 * _l1 + 1):
                for _jl in range(_d2):
                    _c = float(_C[_ii, _jl, _kk])
                    if _c != 0.0:
                        for _u in range(_mul1):
                            _G[_jl * _FPAD + _NODE_OFF[_i1] + _ii * _mul1 + _u,
                               _coff + _kk * _mul1 + _u] = _c
        _coff += (2 * _lo + 1) * _mul1
    _G_MATS.append(_G)
    _cursor += _cols
_TALL_DIM = _cursor

_BT_ROWS, _BT_COLS = [], []
for _gi, (_mul, _l) in enumerate(_IRR_NODE):
    _d = 2 * _l + 1
    for _kk in range(_d):
        for _u in range(_mul):
            for _v in range(_mul):
                _BT_ROWS.append(_NODE_OFF[_gi] + _kk * _mul + _u)
                _BT_COLS.append(_NODE_OFF[_gi] + _v * _d + _kk)
_BT_ROWS = np.asarray(_BT_ROWS, np.int32)
_BT_COLS = np.asarray(_BT_COLS, np.int32)

def _build_bt(lw0, lw1, lw2):
    vals = []
    for lw, (mul, l) in zip((lw0, lw1, lw2), _IRR_NODE):
        d = 2 * l + 1
        vals.append(jnp.tile(lw[None, :, :] / np.float32(math.sqrt(mul)), (d, 1, 1)).reshape(-1))
    vals = jnp.concatenate(vals)
    return jnp.zeros((_FPAD, _FPAD), jnp.float32).at[_BT_ROWS, _BT_COLS].set(vals)


def _silu(x):
    return x / (1.0 + jnp.exp(-x))


def _tc_body(x1_ref, sh_ref, er_ref, w1_ref, b1_ref, w2_ref, b2_ref, w3_ref,
             b3_ref, bt_ref, g0_ref, g1_ref, g2_ref, *rest):
    out_ref = rest[-1]
    rs = rest[:-1]
    f32 = jnp.float32
    bf16 = jnp.bfloat16

    er = er_ref[...]
    h = _silu(jnp.dot(er.astype(bf16), w1_ref[...], preferred_element_type=f32) + b1_ref[0:1, :])
    h = _silu(jnp.dot(h.astype(bf16), w2_ref[...], preferred_element_type=f32) + b2_ref[0:1, :])
    w = (jnp.dot(h.astype(bf16), w3_ref[...], preferred_element_type=f32)
         + b3_ref[0:1, :]).astype(bf16)

    sh = sh_ref[...].astype(bf16)
    x1 = x1_ref[...].astype(bf16)

    zs = [x1 * sh[:, j:j + 1] for j in range(9)]
    g_refs = (g0_ref, g1_ref, g2_ref)
    tparts = []
    for g, (lo_j, hi_j) in enumerate(((0, 1), (1, 4), (4, 9))):
        z = zs[lo_j] if hi_j - lo_j == 1 else jnp.concatenate(zs[lo_j:hi_j], axis=1)
        tparts.append(jnp.dot(z, g_refs[g][...], preferred_element_type=f32).astype(bf16))
    tall = jnp.concatenate(tparts, axis=1)

    parts = [[None] * (2 * l + 1) for (_, l) in _IRR_NODE]
    for idx, (i1, i2, io) in enumerate(_INS):
        mul1, l1 = _IRR_NODE[i1]
        mulo, lo = _IRR_NODE[io]
        nw = mul1 * mulo
        r_ref, s_ref = rs[2 * idx], rs[2 * idx + 1]
        wsl = w[:, _W_OFF[idx]:_W_OFF[idx] + nw]
        for k in range(2 * lo + 1):
            tk = tall[:, _TALL_OFF[idx] + k * mul1:_TALL_OFF[idx] + (k + 1) * mul1]
            texp = jnp.dot(tk, r_ref[...], preferred_element_type=f32)
            part = jnp.dot((wsl * texp).astype(bf16), s_ref[...], preferred_element_type=f32)
            parts[io][k] = part if parts[io][k] is None else parts[io][k] + part

    cols = []
    for gi in range(3):
        cols.extend(parts[gi])
    cols.append(jnp.zeros((x1.shape[0], _FPAD - _DIM_NODE), f32))
    msg_t = jnp.concatenate(cols, axis=1)
    out_ref[...] = jnp.dot(msg_t.astype(bf16), bt_ref[...], preferred_element_type=f32)


def _tc_messages(x1t, sh_pad, er_pad, w1t, b1r, w2t, b2r, w3t, b3r, bt, consts):
    epad = x1t.shape[0]
    grid = (epad // _BE,)
    edge_spec = lambda width: pl.BlockSpec((_BE, width), lambda i: (i, 0))
    full = lambda a: pl.BlockSpec(a.shape, lambda i: (0, 0))
    gmats = [jnp.asarray(g, jnp.bfloat16) for g in _G_MATS]
    in_specs = [edge_spec(_FPAD), edge_spec(16), edge_spec(8),
                full(w1t), full(b1r), full(w2t), full(b2r), full(w3t),
                full(b3r), full(bt)] + [full(g) for g in gmats] + [full(c) for c in consts]
    return pl.pallas_call(
        _tc_body,
        grid=grid,
        in_specs=in_specs,
        out_specs=pl.BlockSpec((_BE, _FPAD), lambda i: (i, 0)),
        out_shape=jax.ShapeDtypeStruct((epad, _FPAD), jnp.float32),
    )(x1t, sh_pad, er_pad, w1t, b1r, w2t, b2r, w3t, b3r, bt, *gmats, *consts)


def _sc_gather(table, src_pad):
    epad = src_pad.shape[0]
    mesh = plsc.VectorSubcoreMesh(core_axis_name="c", subcore_axis_name="s")
    rows_pt = epad // 32
    nch = rows_pt // _CH

    @functools.partial(
        pl.kernel, mesh=mesh,
        out_type=jax.ShapeDtypeStruct((epad, _FPAD), jnp.float32),
        scratch_types=[pltpu.VMEM((_CH,), jnp.int32),
                       pltpu.VMEM((_CH, _FPAD), jnp.float32),
                       pltpu.SemaphoreType.DMA],
    )
    def k(table_hbm, src_hbm, out_hbm, idx_v, rows_v, sem):
        wid = lax.axis_index("s") * 2 + lax.axis_index("c")

        def step(i, carry):
            base = wid * rows_pt + i * _CH
            pltpu.sync_copy(src_hbm.at[pl.ds(base, _CH)], idx_v)
            pltpu.async_copy(table_hbm.at[idx_v], rows_v, sem).wait()
            pltpu.sync_copy(rows_v, out_hbm.at[pl.ds(base, _CH)])
            return carry

        lax.fori_loop(0, nch, step, 0)

    return k(table, src_pad)


def _sc_scatter(msgs, dst_pad, nf_pad):
    epad = msgs.shape[0]
    npad = nf_pad.shape[0]
    mesh = plsc.VectorSubcoreMesh(core_axis_name="c", subcore_axis_name="s")
    rows_pt = epad // 16
    nch = rows_pt // _CH
    init_pt = npad // 16

    @functools.partial(
        pl.kernel, mesh=mesh,
        out_type=jax.ShapeDtypeStruct((npad, _FPAD), jnp.float32),
        scratch_types=[pltpu.VMEM((_CH, _FPAD), jnp.float32),
                       pltpu.VMEM((_CH,), jnp.int32),
                       pltpu.VMEM_SHARED((npad, _FPAD), jnp.float32)],
    )
    def k(msg_hbm, dst_hbm, nf_hbm, out_hbm, buf_v, idx_v, acc_sh):
        cid = lax.axis_index("c")
        sid = lax.axis_index("s")

        @pl.when(cid == 0)
        def _():
            pltpu.sync_copy(nf_hbm.at[pl.ds(sid * init_pt, init_pt)],
                            acc_sh.at[pl.ds(sid * init_pt, init_pt)])
            plsc.subcore_barrier()

            def step(i, carry):
                base = sid * rows_pt + i * _CH
                pltpu.sync_copy(dst_hbm.at[pl.ds(base, _CH)], idx_v)
                pltpu.sync_copy(msg_hbm.at[pl.ds(base, _CH)], buf_v)
                pltpu.sync_copy(buf_v, acc_sh.at[idx_v], add=True)
                return carry

            lax.fori_loop(0, nch, step, 0)
            plsc.subcore_barrier()
            pltpu.sync_copy(acc_sh.at[pl.ds(sid * init_pt, init_pt)],
                            out_hbm.at[pl.ds(sid * init_pt, init_pt)])

    return k(msgs, dst_pad, nf_pad)


def kernel(node_features, edge_index, edge_sh, edge_radial, W1, b1, W2, b2,
           W3, b3, lw0, lw1, lw2):
    N = node_features.shape[0]
    E = edge_sh.shape[0]
    epad = ((E + 4095) // 4096) * 4096
    npad = ((N + 2047) // 2048) * 2048

    segs = []
    for gi, (mul, l) in enumerate(_IRR_NODE):
        d = 2 * l + 1
        p = node_features[:, _NODE_OFF[gi]:_NODE_OFF[gi] + mul * d]
        segs.append(p.reshape(N, mul, d).transpose(0, 2, 1).reshape(N, mul * d))
    node_t = jnp.concatenate(segs + [jnp.zeros((N, _FPAD - _DIM_NODE), jnp.float32)], axis=1)

    src = edge_index[0].astype(jnp.int32)
    dst = edge_index[1].astype(jnp.int32)
    src_pad = jnp.zeros((epad,), jnp.int32).at[:E].set(src)
    dst_pad = jnp.zeros((epad,), jnp.int32).at[:E].set(dst)
    sh_pad = jnp.zeros((epad, 16), jnp.float32).at[:E, :9].set(edge_sh)
    er_pad = jnp.zeros((epad, 8), jnp.float32).at[:E, :].set(edge_radial)

    bf16 = jnp.bfloat16
    w1t = W1.T.astype(bf16)
    w2t = W2.T.astype(bf16)
    w3t = W3.T.astype(bf16)
    b1r = jnp.tile(b1[None, :], (8, 1))
    b2r = jnp.tile(b2[None, :], (8, 1))
    b3r = jnp.tile(b3[None, :], (8, 1))
    bt = _build_bt(lw0, lw1, lw2).astype(bf16)

    consts = []
    for r, s in zip(_R_MATS, _S_MATS):
        consts.append(jnp.asarray(r, bf16))
        consts.append(jnp.asarray(s, bf16))

    x1t = _sc_gather(node_t, src_pad)
    msgs = _tc_messages(x1t, sh_pad, er_pad, w1t, b1r, w2t, b2r, w3t, b3r, bt, consts)

    nf_pad = jnp.zeros((npad, _FPAD), jnp.float32).at[:N, :_DIM_NODE].set(node_features)
    outp = _sc_scatter(msgs, dst_pad, nf_pad)
    return outp[:N, :_DIM_NODE]


# speedup vs baseline: 1.2906x; 1.0821x over previous
import functools
import math

import jax
import jax.numpy as jnp
import numpy as np
from jax import lax
from jax.experimental import pallas as pl
from jax.experimental.pallas import tpu as pltpu
from jax.experimental.pallas import tpu_sc as plsc

_IRR_NODE = ((32, 0), (16, 1), (8, 2))
_IRR_EDGE = ((1, 0), (1, 1), (1, 2))
_INS = ((0, 0, 0), (0, 1, 1), (0, 2, 2), (1, 0, 1), (1, 1, 0), (1, 1, 2),
        (1, 2, 1), (2, 0, 2), (2, 1, 1), (2, 2, 0), (2, 2, 2))


def _su2_cg_coeff(j1, m1, j2, m2, j3, m3):
    if m3 != m1 + m2:
        return 0.0
    f = math.factorial
    vmin = int(max(-j1 + j2 + m3, -j1 + m1, 0))
    vmax = int(min(j2 + j3 + m1, j3 - j1 + j2, j3 + m3))
    C = math.sqrt((2 * j3 + 1) * f(j3 + j1 - j2) * f(j3 - j1 + j2) * f(j1 + j2 - j3) * f(j3 + m3) * f(j3 - m3)
                  / (f(j1 + j2 + j3 + 1) * f(j1 - m1) * f(j1 + m1) * f(j2 - m2) * f(j2 + m2)))
    S = 0.0
    for v in range(vmin, vmax + 1):
        S += (-1.0) ** (v + j2 + m2) * f(j2 + j3 + m1 - v) * f(j1 - m1 + v) / (
            f(v) * f(j3 - j1 + j2 - v) * f(j3 + m3 - v) * f(v + j1 - j2 - m3))
    return C * S


def _real_q(l):
    q = np.zeros((2 * l + 1, 2 * l + 1), dtype=np.complex128)
    for m in range(-l, 0):
        q[l + m, l + abs(m)] = 1.0 / math.sqrt(2.0)
        q[l + m, l - abs(m)] = -1j / math.sqrt(2.0)
    q[l, l] = 1.0
    for m in range(1, l + 1):
        q[l + m, l + abs(m)] = (-1) ** m / math.sqrt(2.0)
        q[l + m, l - abs(m)] = 1j * ((-1) ** m) / math.sqrt(2.0)
    return ((-1j) ** l) * q


def _wigner3j(l1, l2, l3):
    C = np.zeros((2 * l1 + 1, 2 * l2 + 1, 2 * l3 + 1))
    for m1 in range(-l1, l1 + 1):
        for m2 in range(-l2, l2 + 1):
            m3 = m1 + m2
            if abs(m3) <= l3:
                C[l1 + m1, l2 + m2, l3 + m3] = _su2_cg_coeff(l1, m1, l2, m2, l3, m3)
    Q1, Q2, Q3 = _real_q(l1), _real_q(l2), _real_q(l3)
    Cr = np.real(np.einsum('ij,kl,mn,ikn->jlm', Q1, Q2, np.conj(Q3.T), C.astype(np.complex128)))
    n = np.linalg.norm(Cr)
    return (Cr / n).astype(np.float32) if n > 0 else Cr.astype(np.float32)


_C3J = {}
for _a, _b, _c in _INS:
    _k = (_IRR_NODE[_a][1], _IRR_EDGE[_b][1], _IRR_NODE[_c][1])
    if _k not in _C3J:
        _C3J[_k] = _wigner3j(*_k)

_PATHC = []
for (_i1, _i2, _io) in _INS:
    _fan = sum(_IRR_NODE[a][0] * _IRR_EDGE[b][0] for (a, b, c) in _INS if c == _io)
    _PATHC.append(math.sqrt((2 * _IRR_NODE[_io][1] + 1) / _fan))

_NODE_OFF = [0, 32, 80]
_EDGE_OFF = [0, 1, 4]
_DIM_NODE = 120
_W_OFF = []
_off = 0
for (_i1, _i2, _io) in _INS:
    _W_OFF.append(_off)
    _off += _IRR_NODE[_i1][0] * _IRR_NODE[_io][0]
_W_TOT = _off

_NNZ = []
_R_MATS = []
_S_MATS = []
for _idx, (_i1, _i2, _io) in enumerate(_INS):
    _mul1, _l1 = _IRR_NODE[_i1]
    _l2 = _IRR_EDGE[_i2][1]
    _mulo, _lo = _IRR_NODE[_io]
    _C = _C3J[(_l1, _l2, _lo)]
    _per_k = []
    for _kk in range(2 * _lo + 1):
        _lst = []
        for _ii in range(2 * _l1 + 1):
            for _jj in range(2 * _l2 + 1):
                _c = float(_C[_ii, _jj, _kk])
                if _c != 0.0:
                    _lst.append((_ii, _jj, _c))
        _per_k.append(_lst)
    _NNZ.append(_per_k)
    _R = np.zeros((_mul1, _mul1 * _mulo), np.float32)
    _S = np.zeros((_mul1 * _mulo, _mulo), np.float32)
    for _u in range(_mul1):
        for _w in range(_mulo):
            _R[_u, _u * _mulo + _w] = 1.0
            _S[_u * _mulo + _w, _w] = _PATHC[_idx]
    _R_MATS.append(_R)
    _S_MATS.append(_S)

_BE = 1024
_CH = 128
_FPAD = 128

_I2_GROUPS = [[i for i, ins in enumerate(_INS) if ins[1] == g] for g in range(3)]
_TALL_OFF = {}
_G_MATS = []
_cursor = 0
for _g in range(3):
    _d2 = 2 * _IRR_EDGE[_g][1] + 1
    _cols = 0
    for _idx in _I2_GROUPS[_g]:
        _i1 = _INS[_idx][0]
        _lo = _IRR_NODE[_INS[_idx][2]][1]
        _TALL_OFF[_idx] = _cursor + _cols
        _cols += (2 * _lo + 1) * _IRR_NODE[_i1][0]
    _G = np.zeros((_d2 * _FPAD, _cols), np.float32)
    _coff = 0
    for _idx in _I2_GROUPS[_g]:
        _i1, _i2, _io = _INS[_idx]
        _mul1, _l1 = _IRR_NODE[_i1]
        _lo = _IRR_NODE[_io][1]
        _C = _C3J[(_l1, _IRR_EDGE[_i2][1], _lo)]
        for _kk in range(2 * _lo + 1):
            for _ii in range(2 * _l1 + 1):
                for _jl in range(_d2):
                    _c = float(_C[_ii, _jl, _kk])
                    if _c != 0.0:
                        for _u in range(_mul1):
                            _G[_jl * _FPAD + _NODE_OFF[_i1] + _ii * _mul1 + _u,
                               _coff + _kk * _mul1 + _u] = _c
        _coff += (2 * _lo + 1) * _mul1
    _G_MATS.append(_G)
    _cursor += _cols
_TALL_DIM = _cursor

_BT_ROWS, _BT_COLS = [], []
for _gi, (_mul, _l) in enumerate(_IRR_NODE):
    _d = 2 * _l + 1
    for _kk in range(_d):
        for _u in range(_mul):
            for _v in range(_mul):
                _BT_ROWS.append(_NODE_OFF[_gi] + _kk * _mul + _u)
                _BT_COLS.append(_NODE_OFF[_gi] + _v * _d + _kk)
_BT_ROWS = np.asarray(_BT_ROWS, np.int32)
_BT_COLS = np.asarray(_BT_COLS, np.int32)

def _build_bt(lw0, lw1, lw2):
    vals = []
    for lw, (mul, l) in zip((lw0, lw1, lw2), _IRR_NODE):
        d = 2 * l + 1
        vals.append(jnp.tile(lw[None, :, :] / np.float32(math.sqrt(mul)), (d, 1, 1)).reshape(-1))
    vals = jnp.concatenate(vals)
    return jnp.zeros((_FPAD, _FPAD), jnp.float32).at[_BT_ROWS, _BT_COLS].set(vals)


def _silu(x):
    return x / (1.0 + jnp.exp(-x))


def _tc_body(x1_ref, sh_ref, er_ref, w1_ref, b1_ref, w2_ref, b2_ref, w3_ref,
             b3_ref, bt_ref, g0_ref, g1_ref, g2_ref, *rest):
    out_ref = rest[-1]
    rs = rest[:-1]
    f32 = jnp.float32
    bf16 = jnp.bfloat16

    er = er_ref[...]
    h = _silu(jnp.dot(er.astype(bf16), w1_ref[...], preferred_element_type=f32) + b1_ref[0:1, :])
    h = _silu(jnp.dot(h.astype(bf16), w2_ref[...], preferred_element_type=f32) + b2_ref[0:1, :])
    w = (jnp.dot(h.astype(bf16), w3_ref[...], preferred_element_type=f32)
         + b3_ref[0:1, :]).astype(bf16)

    sh = sh_ref[...].astype(bf16)
    x1 = x1_ref[...].astype(bf16)

    zs = [x1 * sh[:, j:j + 1] for j in range(9)]
    g_refs = (g0_ref, g1_ref, g2_ref)
    tparts = []
    for g, (lo_j, hi_j) in enumerate(((0, 1), (1, 4), (4, 9))):
        z = zs[lo_j] if hi_j - lo_j == 1 else jnp.concatenate(zs[lo_j:hi_j], axis=1)
        tparts.append(jnp.dot(z, g_refs[g][...], preferred_element_type=f32).astype(bf16))
    tall = jnp.concatenate(tparts, axis=1)

    parts = [[None] * (2 * l + 1) for (_, l) in _IRR_NODE]
    for idx, (i1, i2, io) in enumerate(_INS):
        mul1, l1 = _IRR_NODE[i1]
        mulo, lo = _IRR_NODE[io]
        nw = mul1 * mulo
        r_ref, s_ref = rs[2 * idx], rs[2 * idx + 1]
        wsl = w[:, _W_OFF[idx]:_W_OFF[idx] + nw]
        for k in range(2 * lo + 1):
            tk = tall[:, _TALL_OFF[idx] + k * mul1:_TALL_OFF[idx] + (k + 1) * mul1]
            texp = jnp.dot(tk, r_ref[...], preferred_element_type=f32)
            part = jnp.dot((wsl * texp).astype(bf16), s_ref[...], preferred_element_type=f32)
            parts[io][k] = part if parts[io][k] is None else parts[io][k] + part

    cols = []
    for gi in range(3):
        cols.extend(parts[gi])
    cols.append(jnp.zeros((x1.shape[0], _FPAD - _DIM_NODE), f32))
    msg_t = jnp.concatenate(cols, axis=1)
    out_ref[...] = jnp.dot(msg_t.astype(bf16), bt_ref[...], preferred_element_type=f32)


def _tc_messages(x1t, sh_pad, er_pad, w1t, b1r, w2t, b2r, w3t, b3r, bt, consts):
    epad = x1t.shape[0]
    grid = (epad // _BE,)
    edge_spec = lambda width: pl.BlockSpec((_BE, width), lambda i: (i, 0))
    full = lambda a: pl.BlockSpec(a.shape, lambda i: (0, 0))
    gmats = [jnp.asarray(g, jnp.bfloat16) for g in _G_MATS]
    in_specs = [edge_spec(_FPAD), edge_spec(16), edge_spec(8),
                full(w1t), full(b1r), full(w2t), full(b2r), full(w3t),
                full(b3r), full(bt)] + [full(g) for g in gmats] + [full(c) for c in consts]
    return pl.pallas_call(
        _tc_body,
        grid=grid,
        in_specs=in_specs,
        out_specs=pl.BlockSpec((_BE, _FPAD), lambda i: (i, 0)),
        out_shape=jax.ShapeDtypeStruct((epad, _FPAD), jnp.float32),
    )(x1t, sh_pad, er_pad, w1t, b1r, w2t, b2r, w3t, b3r, bt, *gmats, *consts)


def _sc_gather(table, src_pad):
    epad = src_pad.shape[0]
    mesh = plsc.VectorSubcoreMesh(core_axis_name="c", subcore_axis_name="s")
    rows_pt = epad // 32
    nch = rows_pt // _CH

    @functools.partial(
        pl.kernel, mesh=mesh,
        out_type=jax.ShapeDtypeStruct((epad, _FPAD), jnp.float32),
        scratch_types=[pltpu.VMEM((_CH,), jnp.int32),
                       pltpu.VMEM((_CH, _FPAD), jnp.float32),
                       pltpu.SemaphoreType.DMA],
    )
    def k(table_hbm, src_hbm, out_hbm, idx_v, rows_v, sem):
        wid = lax.axis_index("s") * 2 + lax.axis_index("c")

        def step(i, carry):
            base = wid * rows_pt + i * _CH
            pltpu.sync_copy(src_hbm.at[pl.ds(base, _CH)], idx_v)
            pltpu.async_copy(table_hbm.at[idx_v], rows_v, sem).wait()
            pltpu.sync_copy(rows_v, out_hbm.at[pl.ds(base, _CH)])
            return carry

        lax.fori_loop(0, nch, step, 0)

    return k(table, src_pad)


def _sc_scatter(msgs, dst_pad, nf_pad):
    epad = msgs.shape[0]
    npad = nf_pad.shape[0]
    mesh = plsc.VectorSubcoreMesh(core_axis_name="c", subcore_axis_name="s")
    rows_pt = epad // 16
    nch = rows_pt // _CH
    init_pt = npad // 16

    @functools.partial(
        pl.kernel, mesh=mesh,
        out_type=jax.ShapeDtypeStruct((npad, _FPAD), jnp.float32),
        scratch_types=[pltpu.VMEM((_CH, _FPAD), jnp.float32),
                       pltpu.VMEM((_CH,), jnp.int32),
                       pltpu.VMEM_SHARED((npad, _FPAD), jnp.float32)],
    )
    def k(msg_hbm, dst_hbm, nf_hbm, out_hbm, buf_v, idx_v, acc_sh):
        cid = lax.axis_index("c")
        sid = lax.axis_index("s")

        @pl.when(cid == 0)
        def _():
            pltpu.sync_copy(nf_hbm.at[pl.ds(sid * init_pt, init_pt)],
                            acc_sh.at[pl.ds(sid * init_pt, init_pt)])
            plsc.subcore_barrier()

            def step(i, carry):
                base = sid * rows_pt + i * _CH
                pltpu.sync_copy(dst_hbm.at[pl.ds(base, _CH)], idx_v)
                pltpu.sync_copy(msg_hbm.at[pl.ds(base, _CH)], buf_v)
                pltpu.sync_copy(buf_v, acc_sh.at[idx_v], add=True)
                return carry

            lax.fori_loop(0, nch, step, 0)
            plsc.subcore_barrier()
            pltpu.sync_copy(acc_sh.at[pl.ds(sid * init_pt, init_pt)],
                            out_hbm.at[pl.ds(sid * init_pt, init_pt)])

    return k(msgs, dst_pad, nf_pad)


def kernel(node_features, edge_index, edge_sh, edge_radial, W1, b1, W2, b2,
           W3, b3, lw0, lw1, lw2):
    N = node_features.shape[0]
    E = edge_sh.shape[0]
    epad = ((E + 4095) // 4096) * 4096
    npad = ((N + 2047) // 2048) * 2048

    segs = []
    for gi, (mul, l) in enumerate(_IRR_NODE):
        d = 2 * l + 1
        p = node_features[:, _NODE_OFF[gi]:_NODE_OFF[gi] + mul * d]
        segs.append(p.reshape(N, mul, d).transpose(0, 2, 1).reshape(N, mul * d))
    node_t = jnp.concatenate(segs + [jnp.zeros((N, _FPAD - _DIM_NODE), jnp.float32)], axis=1)

    src = edge_index[0].astype(jnp.int32)
    dst = edge_index[1].astype(jnp.int32)
    src_pad = jnp.zeros((epad,), jnp.int32).at[:E].set(src)
    dst_pad = jnp.zeros((epad,), jnp.int32).at[:E].set(dst)
    sh_pad = jnp.zeros((epad, 16), jnp.float32).at[:E, :9].set(edge_sh)
    er_pad = jnp.zeros((epad, 8), jnp.float32).at[:E, :].set(edge_radial)

    bf16 = jnp.bfloat16
    w1t = W1.T.astype(bf16)
    w2t = W2.T.astype(bf16)
    w3t = W3.T.astype(bf16)
    b1r = jnp.tile(b1[None, :], (8, 1))
    b2r = jnp.tile(b2[None, :], (8, 1))
    b3r = jnp.tile(b3[None, :], (8, 1))
    bt = _build_bt(lw0, lw1, lw2).astype(bf16)

    consts = []
    for r, s in zip(_R_MATS, _S_MATS):
        consts.append(jnp.asarray(r, bf16))
        consts.append(jnp.asarray(s, bf16))

    x1t = _sc_gather(node_t, src_pad)
    msgs = _tc_messages(x1t, sh_pad, er_pad, w1t, b1r, w2t, b2r, w3t, b3r, bt, consts)

    nf_pad = jnp.zeros((npad, _FPAD), jnp.float32).at[:N, :_DIM_NODE].set(node_features)
    outp = _sc_scatter(msgs, dst_pad, nf_pad)
    return outp[:N, :_DIM_NODE]


# speedup vs baseline: 1.3082x; 1.0136x over previous
import functools
import math

import jax
import jax.numpy as jnp
import numpy as np
from jax import lax
from jax.experimental import pallas as pl
from jax.experimental.pallas import tpu as pltpu
from jax.experimental.pallas import tpu_sc as plsc

_IRR_NODE = ((32, 0), (16, 1), (8, 2))
_IRR_EDGE = ((1, 0), (1, 1), (1, 2))
_INS = ((0, 0, 0), (0, 1, 1), (0, 2, 2), (1, 0, 1), (1, 1, 0), (1, 1, 2),
        (1, 2, 1), (2, 0, 2), (2, 1, 1), (2, 2, 0), (2, 2, 2))


def _su2_cg_coeff(j1, m1, j2, m2, j3, m3):
    if m3 != m1 + m2:
        return 0.0
    f = math.factorial
    vmin = int(max(-j1 + j2 + m3, -j1 + m1, 0))
    vmax = int(min(j2 + j3 + m1, j3 - j1 + j2, j3 + m3))
    C = math.sqrt((2 * j3 + 1) * f(j3 + j1 - j2) * f(j3 - j1 + j2) * f(j1 + j2 - j3) * f(j3 + m3) * f(j3 - m3)
                  / (f(j1 + j2 + j3 + 1) * f(j1 - m1) * f(j1 + m1) * f(j2 - m2) * f(j2 + m2)))
    S = 0.0
    for v in range(vmin, vmax + 1):
        S += (-1.0) ** (v + j2 + m2) * f(j2 + j3 + m1 - v) * f(j1 - m1 + v) / (
            f(v) * f(j3 - j1 + j2 - v) * f(j3 + m3 - v) * f(v + j1 - j2 - m3))
    return C * S


def _real_q(l):
    q = np.zeros((2 * l + 1, 2 * l + 1), dtype=np.complex128)
    for m in range(-l, 0):
        q[l + m, l + abs(m)] = 1.0 / math.sqrt(2.0)
        q[l + m, l - abs(m)] = -1j / math.sqrt(2.0)
    q[l, l] = 1.0
    for m in range(1, l + 1):
        q[l + m, l + abs(m)] = (-1) ** m / math.sqrt(2.0)
        q[l + m, l - abs(m)] = 1j * ((-1) ** m) / math.sqrt(2.0)
    return ((-1j) ** l) * q


def _wigner3j(l1, l2, l3):
    C = np.zeros((2 * l1 + 1, 2 * l2 + 1, 2 * l3 + 1))
    for m1 in range(-l1, l1 + 1):
        for m2 in range(-l2, l2 + 1):
            m3 = m1 + m2
            if abs(m3) <= l3:
                C[l1 + m1, l2 + m2, l3 + m3] = _su2_cg_coeff(l1, m1, l2, m2, l3, m3)
    Q1, Q2, Q3 = _real_q(l1), _real_q(l2), _real_q(l3)
    Cr = np.real(np.einsum('ij,kl,mn,ikn->jlm', Q1, Q2, np.conj(Q3.T), C.astype(np.complex128)))
    n = np.linalg.norm(Cr)
    return (Cr / n).astype(np.float32) if n > 0 else Cr.astype(np.float32)


_C3J = {}
for _a, _b, _c in _INS:
    _k = (_IRR_NODE[_a][1], _IRR_EDGE[_b][1], _IRR_NODE[_c][1])
    if _k not in _C3J:
        _C3J[_k] = _wigner3j(*_k)

_PATHC = []
for (_i1, _i2, _io) in _INS:
    _fan = sum(_IRR_NODE[a][0] * _IRR_EDGE[b][0] for (a, b, c) in _INS if c == _io)
    _PATHC.append(math.sqrt((2 * _IRR_NODE[_io][1] + 1) / _fan))

_NODE_OFF = [0, 32, 80]
_EDGE_OFF = [0, 1, 4]
_DIM_NODE = 120
_W_OFF = []
_off = 0
for (_i1, _i2, _io) in _INS:
    _W_OFF.append(_off)
    _off += _IRR_NODE[_i1][0] * _IRR_NODE[_io][0]
_W_TOT = _off

_NNZ = []
_R_MATS = []
_S_MATS = []
for _idx, (_i1, _i2, _io) in enumerate(_INS):
    _mul1, _l1 = _IRR_NODE[_i1]
    _l2 = _IRR_EDGE[_i2][1]
    _mulo, _lo = _IRR_NODE[_io]
    _C = _C3J[(_l1, _l2, _lo)]
    _per_k = []
    for _kk in range(2 * _lo + 1):
        _lst = []
        for _ii in range(2 * _l1 + 1):
            for _jj in range(2 * _l2 + 1):
                _c = float(_C[_ii, _jj, _kk])
                if _c != 0.0:
                    _lst.append((_ii, _jj, _c))
        _per_k.append(_lst)
    _NNZ.append(_per_k)
    _R = np.zeros((_mul1, _mul1 * _mulo), np.float32)
    _S = np.zeros((_mul1 * _mulo, _mulo), np.float32)
    for _u in range(_mul1):
        for _w in range(_mulo):
            _R[_u, _u * _mulo + _w] = 1.0
            _S[_u * _mulo + _w, _w] = _PATHC[_idx]
    _R_MATS.append(_R)
    _S_MATS.append(_S)

_BE = 1024
_CH = 128
_FPAD = 128

_I2_GROUPS = [[i for i, ins in enumerate(_INS) if ins[1] == g] for g in range(3)]
_TALL_OFF = {}
_G_MATS = []
_cursor = 0
for _g in range(3):
    _d2 = 2 * _IRR_EDGE[_g][1] + 1
    _cols = 0
    for _idx in _I2_GROUPS[_g]:
        _i1 = _INS[_idx][0]
        _lo = _IRR_NODE[_INS[_idx][2]][1]
        _TALL_OFF[_idx] = _cursor + _cols
        _cols += (2 * _lo + 1) * _IRR_NODE[_i1][0]
    _G = np.zeros((_d2 * _FPAD, _cols), np.float32)
    _coff = 0
    for _idx in _I2_GROUPS[_g]:
        _i1, _i2, _io = _INS[_idx]
        _mul1, _l1 = _IRR_NODE[_i1]
        _lo = _IRR_NODE[_io][1]
        _C = _C3J[(_l1, _IRR_EDGE[_i2][1], _lo)]
        for _kk in range(2 * _lo + 1):
            for _ii in range(2 * _l1 + 1):
                for _jl in range(_d2):
                    _c = float(_C[_ii, _jl, _kk])
                    if _c != 0.0:
                        for _u in range(_mul1):
                            _G[_jl * _FPAD + _NODE_OFF[_i1] + _ii * _mul1 + _u,
                               _coff + _kk * _mul1 + _u] = _c
        _coff += (2 * _lo + 1) * _mul1
    _G_MATS.append(_G)
    _cursor += _cols
_TALL_DIM = _cursor

_BT_ROWS, _BT_COLS = [], []
for _gi, (_mul, _l) in enumerate(_IRR_NODE):
    _d = 2 * _l + 1
    for _kk in range(_d):
        for _u in range(_mul):
            for _v in range(_mul):
                _BT_ROWS.append(_NODE_OFF[_gi] + _kk * _mul + _u)
                _BT_COLS.append(_NODE_OFF[_gi] + _v * _d + _kk)
_BT_ROWS = np.asarray(_BT_ROWS, np.int32)
_BT_COLS = np.asarray(_BT_COLS, np.int32)

def _build_bt(lw0, lw1, lw2):
    vals = []
    for lw, (mul, l) in zip((lw0, lw1, lw2), _IRR_NODE):
        d = 2 * l + 1
        vals.append(jnp.tile(lw[None, :, :] / np.float32(math.sqrt(mul)), (d, 1, 1)).reshape(-1))
    vals = jnp.concatenate(vals)
    return jnp.zeros((_FPAD, _FPAD), jnp.float32).at[_BT_ROWS, _BT_COLS].set(vals)


def _silu(x):
    return x / (1.0 + jnp.exp(-x))


def _tc_body(x1_ref, sh_ref, er_ref, w1_ref, b1_ref, w2_ref, b2_ref, w3_ref,
             b3_ref, bt_ref, g0_ref, g1_ref, g2_ref, *rest):
    out_ref = rest[-1]
    rs = rest[:-1]
    f32 = jnp.float32
    bf16 = jnp.bfloat16

    er = er_ref[...]
    h = _silu(jnp.dot(er.astype(bf16), w1_ref[...], preferred_element_type=f32) + b1_ref[0:1, :])
    h = _silu(jnp.dot(h.astype(bf16), w2_ref[...], preferred_element_type=f32) + b2_ref[0:1, :])
    w = (jnp.dot(h.astype(bf16), w3_ref[...], preferred_element_type=f32)
         + b3_ref[0:1, :]).astype(bf16)

    sh = sh_ref[...].astype(bf16)
    x1 = x1_ref[...].astype(bf16)

    zs = [x1 * sh[:, j:j + 1] for j in range(9)]
    g_refs = (g0_ref, g1_ref, g2_ref)
    tparts = []
    for g, (lo_j, hi_j) in enumerate(((0, 1), (1, 4), (4, 9))):
        z = zs[lo_j] if hi_j - lo_j == 1 else jnp.concatenate(zs[lo_j:hi_j], axis=1)
        tparts.append(jnp.dot(z, g_refs[g][...], preferred_element_type=f32).astype(bf16))
    tall = jnp.concatenate(tparts, axis=1)

    parts = [[None] * (2 * l + 1) for (_, l) in _IRR_NODE]
    for idx, (i1, i2, io) in enumerate(_INS):
        mul1, l1 = _IRR_NODE[i1]
        mulo, lo = _IRR_NODE[io]
        nw = mul1 * mulo
        r_ref, s_ref = rs[2 * idx], rs[2 * idx + 1]
        wsl = w[:, _W_OFF[idx]:_W_OFF[idx] + nw]
        for k in range(2 * lo + 1):
            tk = tall[:, _TALL_OFF[idx] + k * mul1:_TALL_OFF[idx] + (k + 1) * mul1]
            texp = jnp.dot(tk, r_ref[...], preferred_element_type=f32)
            part = jnp.dot((wsl * texp).astype(bf16), s_ref[...], preferred_element_type=f32)
            parts[io][k] = part if parts[io][k] is None else parts[io][k] + part

    cols = []
    for gi in range(3):
        cols.extend(parts[gi])
    cols.append(jnp.zeros((x1.shape[0], _FPAD - _DIM_NODE), f32))
    msg_t = jnp.concatenate(cols, axis=1)
    out_ref[...] = jnp.dot(msg_t.astype(bf16), bt_ref[...], preferred_element_type=f32)


def _tc_messages(x1t, sh_pad, er_pad, w1t, b1r, w2t, b2r, w3t, b3r, bt, consts):
    epad = x1t.shape[0]
    grid = (epad // _BE,)
    edge_spec = lambda width: pl.BlockSpec((_BE, width), lambda i: (i, 0))
    full = lambda a: pl.BlockSpec(a.shape, lambda i: (0, 0))
    gmats = [jnp.asarray(g, jnp.bfloat16) for g in _G_MATS]
    in_specs = [edge_spec(_FPAD), edge_spec(16), edge_spec(8),
                full(w1t), full(b1r), full(w2t), full(b2r), full(w3t),
                full(b3r), full(bt)] + [full(g) for g in gmats] + [full(c) for c in consts]
    return pl.pallas_call(
        _tc_body,
        grid=grid,
        in_specs=in_specs,
        out_specs=pl.BlockSpec((_BE, _FPAD), lambda i: (i, 0)),
        out_shape=jax.ShapeDtypeStruct((epad, _FPAD), jnp.float32),
    )(x1t, sh_pad, er_pad, w1t, b1r, w2t, b2r, w3t, b3r, bt, *gmats, *consts)


def _sc_gather(table, src_pad):
    epad = src_pad.shape[0]
    mesh = plsc.VectorSubcoreMesh(core_axis_name="c", subcore_axis_name="s")
    rows_pt = epad // 32
    nch = rows_pt // _CH

    assert nch % 2 == 1 and nch >= 3

    @functools.partial(
        pl.kernel, mesh=mesh,
        out_type=jax.ShapeDtypeStruct((epad, _FPAD), jnp.float32),
        scratch_types=[pltpu.VMEM((_CH,), jnp.int32),
                       pltpu.VMEM((_CH,), jnp.int32),
                       pltpu.VMEM((_CH, _FPAD), jnp.float32),
                       pltpu.VMEM((_CH, _FPAD), jnp.float32),
                       pltpu.SemaphoreType.DMA,
                       pltpu.SemaphoreType.DMA],
    )
    def k(table_hbm, src_hbm, out_hbm, idx_v0, idx_v1, rows_v0, rows_v1,
          sem0, sem1):
        wid = lax.axis_index("s") * 2 + lax.axis_index("c")
        tbase = wid * rows_pt

        def start(c, idx_v, rows_v, sem):
            pltpu.sync_copy(src_hbm.at[pl.ds(tbase + c * _CH, _CH)], idx_v)
            pltpu.async_copy(table_hbm.at[idx_v], rows_v, sem)

        def drain_store(c, idx_v, rows_v, sem):
            pltpu.make_async_copy(table_hbm.at[idx_v], rows_v, sem).wait()
            pltpu.sync_copy(rows_v, out_hbm.at[pl.ds(tbase + c * _CH, _CH)])

        start(0, idx_v0, rows_v0, sem0)

        def step(t, carry):
            c = 2 * t
            start(c + 1, idx_v1, rows_v1, sem1)
            drain_store(c, idx_v0, rows_v0, sem0)
            start(c + 2, idx_v0, rows_v0, sem0)
            drain_store(c + 1, idx_v1, rows_v1, sem1)
            return carry

        lax.fori_loop(0, (nch - 1) // 2, step, 0)
        drain_store(nch - 1, idx_v0, rows_v0, sem0)

    return k(table, src_pad)


def _sc_scatter(msgs, dst_pad, nf_pad):
    epad = msgs.shape[0]
    npad = nf_pad.shape[0]
    mesh = plsc.VectorSubcoreMesh(core_axis_name="c", subcore_axis_name="s")
    rows_pt = epad // 16
    nch = rows_pt // _CH
    init_pt = npad // 16

    @functools.partial(
        pl.kernel, mesh=mesh,
        out_type=jax.ShapeDtypeStruct((npad, _FPAD), jnp.float32),
        scratch_types=[pltpu.VMEM((_CH, _FPAD), jnp.float32),
                       pltpu.VMEM((_CH,), jnp.int32),
                       pltpu.VMEM_SHARED((npad, _FPAD), jnp.float32)],
    )
    def k(msg_hbm, dst_hbm, nf_hbm, out_hbm, buf_v, idx_v, acc_sh):
        cid = lax.axis_index("c")
        sid = lax.axis_index("s")

        @pl.when(cid == 0)
        def _():
            pltpu.sync_copy(nf_hbm.at[pl.ds(sid * init_pt, init_pt)],
                            acc_sh.at[pl.ds(sid * init_pt, init_pt)])
            plsc.subcore_barrier()

            def step(i, carry):
                base = sid * rows_pt + i * _CH
                pltpu.sync_copy(dst_hbm.at[pl.ds(base, _CH)], idx_v)
                pltpu.sync_copy(msg_hbm.at[pl.ds(base, _CH)], buf_v)
                pltpu.sync_copy(buf_v, acc_sh.at[idx_v], add=True)
                return carry

            lax.fori_loop(0, nch, step, 0)
            plsc.subcore_barrier()
            pltpu.sync_copy(acc_sh.at[pl.ds(sid * init_pt, init_pt)],
                            out_hbm.at[pl.ds(sid * init_pt, init_pt)])

    return k(msgs, dst_pad, nf_pad)


def kernel(node_features, edge_index, edge_sh, edge_radial, W1, b1, W2, b2,
           W3, b3, lw0, lw1, lw2):
    N = node_features.shape[0]
    E = edge_sh.shape[0]
    epad = ((E + 4095) // 4096) * 4096
    npad = ((N + 2047) // 2048) * 2048

    segs = []
    for gi, (mul, l) in enumerate(_IRR_NODE):
        d = 2 * l + 1
        p = node_features[:, _NODE_OFF[gi]:_NODE_OFF[gi] + mul * d]
        segs.append(p.reshape(N, mul, d).transpose(0, 2, 1).reshape(N, mul * d))
    node_t = jnp.concatenate(segs + [jnp.zeros((N, _FPAD - _DIM_NODE), jnp.float32)], axis=1)

    src = edge_index[0].astype(jnp.int32)
    dst = edge_index[1].astype(jnp.int32)
    src_pad = jnp.zeros((epad,), jnp.int32).at[:E].set(src)
    dst_pad = jnp.zeros((epad,), jnp.int32).at[:E].set(dst)
    sh_pad = jnp.zeros((epad, 16), jnp.float32).at[:E, :9].set(edge_sh)
    er_pad = jnp.zeros((epad, 8), jnp.float32).at[:E, :].set(edge_radial)

    bf16 = jnp.bfloat16
    w1t = W1.T.astype(bf16)
    w2t = W2.T.astype(bf16)
    w3t = W3.T.astype(bf16)
    b1r = jnp.tile(b1[None, :], (8, 1))
    b2r = jnp.tile(b2[None, :], (8, 1))
    b3r = jnp.tile(b3[None, :], (8, 1))
    bt = _build_bt(lw0, lw1, lw2).astype(bf16)

    consts = []
    for r, s in zip(_R_MATS, _S_MATS):
        consts.append(jnp.asarray(r, bf16))
        consts.append(jnp.asarray(s, bf16))

    x1t = _sc_gather(node_t, src_pad)
    msgs = _tc_messages(x1t, sh_pad, er_pad, w1t, b1r, w2t, b2r, w3t, b3r, bt, consts)

    nf_pad = jnp.zeros((npad, _FPAD), jnp.float32).at[:N, :_DIM_NODE].set(node_features)
    outp = _sc_scatter(msgs, dst_pad, nf_pad)
    return outp[:N, :_DIM_NODE]
